# baseline (speedup 1.0000x reference)
import functools

import jax
import jax.numpy as jnp
from jax import lax
from jax.experimental import pallas as pl
from jax.experimental.pallas import tpu as pltpu, tpu_sc as plsc

N = 50000
D = 128
E = 200000
NREL = 3

NC = 2
NS = 16
L = 16

WSZ = 1024
NWIN = 196
E_PAD = NWIN * WSZ
PADV = N

NPD = 50048
SD = NPD // NS

CS = 5120
NCHUNK = 10
NPA = CS * NCHUNK
SA = CS // NS
NCC = NCHUNK // NC
ER = E_PAD // 128

@functools.cache
def _get_mesh():
    return plsc.VectorSubcoreMesh(core_axis_name="c", subcore_axis_name="s",
                                  num_cores=NC, num_subcores=NS)


WROWS = WSZ // 128


DH = 8
ZB = 4096


def _deg_body(s0, d0, s1, d1, s2, d2, deg_out, win, win8, ones_b, zeros_b,
              colbuf, table):
    c = lax.axis_index("c")
    s = lax.axis_index("s")
    w = s * NC + c

    one16 = jnp.full((L,), 1.0, jnp.float32)
    zero16 = jnp.zeros((L,), jnp.float32)

    def init_ones(i, _):
        ones_b[pl.ds(i * L, L)] = one16
        return _
    lax.fori_loop(0, 128 // L, init_ones, None)

    def init_zeros(i, _):
        zeros_b[pl.ds(i * L, L)] = zero16
        return _
    lax.fori_loop(0, ZB // L, init_zeros, None)

    base = s * SD * DH
    swords = SD * DH

    def zero_stripe():
        for t in range(swords // ZB):
            pltpu.sync_copy(zeros_b, table.at[pl.ds(base + t * ZB, ZB)])
        rem = swords - (swords // ZB) * ZB
        pltpu.sync_copy(zeros_b.at[pl.ds(0, rem)],
                        table.at[pl.ds(base + (swords // ZB) * ZB, rem)])

    zero_stripe()
    plsc.subcore_barrier()

    for h in range(6):
        e = (s0, d0, s1, d1, s2, d2)[h]

        def wbody(i, _):
            wi = w + NC * NS * i

            @pl.when(wi < NWIN)
            def _():
                pltpu.sync_copy(e.at[pl.ds(wi * WROWS, WROWS)], win)
                for g in range(WROWS):
                    def jbody(jj, _):
                        col = jj * L
                        win8[g, pl.ds(col, L)] = win[g, pl.ds(col, L)] * DH
                        return _
                    lax.fori_loop(0, 128 // L, jbody, None)
                    pltpu.sync_copy(ones_b, table.at[win8.at[g]], add=True)
            return _

        lax.fori_loop(0, (NWIN + NC * NS - 1) // (NC * NS), wbody, None)
        plsc.subcore_barrier()

        pltpu.sync_copy(table.at[pl.ds(base, swords)], colbuf)
        pltpu.sync_copy(
            colbuf, deg_out.at[pl.ds((c * 6 + h) * NPD * DH + base, swords)])

        if h < 5:
            zero_stripe()
        plsc.subcore_barrier()


@functools.cache
def _deg_call():
    return pl.kernel(
        _deg_body,
        out_type=jax.ShapeDtypeStruct((NC * 6 * NPD * DH,), jnp.float32),
        mesh=_get_mesh(),
        scratch_types=[
            pltpu.VMEM((WROWS, 128), jnp.int32),
            pltpu.VMEM((WROWS, 128), jnp.int32),
            pltpu.VMEM((128,), jnp.float32),
            pltpu.VMEM((ZB,), jnp.float32),
            pltpu.VMEM((SD * DH,), jnp.float32),
            pltpu.VMEM_SHARED((NPD * DH,), jnp.float32),
        ],
    )


def _scatter_body(y, esall, edall, acc, winS, winD,
                  gidx3, lidx3, rows, zrows, wb, sem, table):
    c = lax.axis_index("c")
    s = lax.axis_index("s")

    zero16 = jnp.zeros((L,), jnp.float32)

    def zinit(i, _):
        r0 = i // (D // L)
        col = (i % (D // L)) * L
        zrows[r0, pl.ds(col, L)] = zero16
        return _
    lax.fori_loop(0, 128 * (D // L), zinit, None)

    iota16 = lax.iota(jnp.int32, L)
    base = s * SA

    def pbody(p, _):
        r = p // NCC
        cc = p % NCC
        lo = (c + NC * cc) * CS
        erow = r * ER

        for t in range(SA // 128):
            pltpu.sync_copy(zrows, table.at[pl.ds(base + t * 128, 128)])
        pltpu.sync_copy(zrows.at[pl.ds(0, SA - (SA // 128) * 128)],
                        table.at[pl.ds(base + (SA // 128) * 128,
                                       SA - (SA // 128) * 128)])
        plsc.subcore_barrier()

        def wbody(i, _):
            wi = s + NS * i

            @pl.when(wi < NWIN)
            def _():
                pltpu.sync_copy(
                    esall.at[pl.ds(erow + wi * WROWS, WROWS)], winS)
                pltpu.sync_copy(
                    edall.at[pl.ds(erow + wi * WROWS, WROWS)], winD)

                for g in range(WROWS):
                    def jbody(jj, _):
                        col = jj * L
                        src16 = winS[g, pl.ds(col, L)]
                        dst16 = winD[g, pl.ds(col, L)]
                        m = (dst16 >= lo) & (dst16 < lo + CS)
                        lidx3[g, pl.ds(col, L)] = jnp.where(
                            m, dst16 - lo, CS + iota16)
                        gidx3[g, pl.ds(col, L)] = src16 + r * N
                        return _
                    lax.fori_loop(0, 128 // L, jbody, None)

                    pltpu.async_copy(y.at[gidx3.at[g]], rows, sem).wait()
                    pltpu.sync_copy(rows, table.at[lidx3.at[g]], add=True)
            return _

        lax.fori_loop(0, (NWIN + NS - 1) // NS, wbody, None)
        plsc.subcore_barrier()

        pltpu.sync_copy(table.at[pl.ds(base, SA)], wb)
        pltpu.sync_copy(wb, acc.at[r, pl.ds(lo + base, SA)])
        plsc.subcore_barrier()
        return _

    lax.fori_loop(0, NREL * NCC, pbody, None)


@functools.cache
def _scatter_call():
    return pl.kernel(
        _scatter_body,
        out_type=jax.ShapeDtypeStruct((NREL, NPA, D), jnp.float32),
        mesh=_get_mesh(),
        scratch_types=[
            pltpu.VMEM((WROWS, 128), jnp.int32),
            pltpu.VMEM((WROWS, 128), jnp.int32),
            pltpu.VMEM((WROWS, 128), jnp.int32),
            pltpu.VMEM((WROWS, 128), jnp.int32),
            pltpu.VMEM((128, D), jnp.float32),
            pltpu.VMEM((128, D), jnp.float32),
            pltpu.VMEM((SA, D), jnp.float32),
            pltpu.SemaphoreType.DMA,
            pltpu.VMEM_SHARED((CS + L, D), jnp.float32),
        ],
    )

MMBLK = 1000


def _make_mm():
    nb = N // MMBLK

    def gen_body(degT_ref, x_ref, w_ref, y_ref):
        r = pl.program_id(0)
        dp = degT_ref[...]
        c0 = dp[:, 0:1] + dp[:, 6:7]
        c1 = dp[:, 2:3] + dp[:, 8:9]
        c2 = dp[:, 4:5] + dp[:, 10:11]
        dsum = jnp.where(r == 0, c0, jnp.where(r == 1, c1, c2))
        norm = jnp.where(dsum > 0, lax.rsqrt(jnp.where(dsum > 0, dsum, 1.0)),
                         0.0)
        y_ref[...] = jnp.dot(x_ref[...] * norm, w_ref[0],
                             preferred_element_type=jnp.float32)

    return pl.pallas_call(
        gen_body,
        grid=(NREL, nb),
        in_specs=[
            pl.BlockSpec((MMBLK, 12), lambda r, b: (b, 0)),
            pl.BlockSpec((MMBLK, D), lambda r, b: (b, 0)),
            pl.BlockSpec((1, D, D), lambda r, b: (r, 0, 0)),
        ],
        out_specs=pl.BlockSpec((MMBLK, D), lambda r, b: (r * nb + b, 0)),
        out_shape=jax.ShapeDtypeStruct((NREL * N, D), jnp.float32),
    )


_mm_call = _make_mm()


def _comb_body(acc_ref, degT_ref, out_ref):
    dp = degT_ref[...]
    out = jnp.zeros((MMBLK, D), jnp.float32)
    for r in range(NREL):
        dsum = dp[:, 2 * r + 1:2 * r + 2] + dp[:, 6 + 2 * r + 1:6 + 2 * r + 2]
        norm = jnp.where(dsum > 0, lax.rsqrt(jnp.where(dsum > 0, dsum, 1.0)),
                         0.0)
        out = out + acc_ref[r] * norm
    out_ref[...] = out


_comb_call = pl.pallas_call(
    _comb_body,
    grid=(N // MMBLK,),
    in_specs=[
        pl.BlockSpec((NREL, MMBLK, D), lambda b: (0, b, 0)),
        pl.BlockSpec((MMBLK, 12), lambda b: (b, 0)),
    ],
    out_specs=pl.BlockSpec((MMBLK, D), lambda b: (b, 0)),
    out_shape=jax.ShapeDtypeStruct((N, D), jnp.float32),
)


def kernel(x, W0, W1, W2, edge_index_0, edge_index_1, edge_index_2):
    pad = ((0, 0), (0, E_PAD - E))
    e0 = jnp.pad(edge_index_0, pad, constant_values=PADV)
    e1 = jnp.pad(edge_index_1, pad, constant_values=PADV)
    e2 = jnp.pad(edge_index_2, pad, constant_values=PADV)
    s0, d0 = e0[0].reshape(ER, 128), e0[1].reshape(ER, 128)
    s1, d1 = e1[0].reshape(ER, 128), e1[1].reshape(ER, 128)
    s2, d2 = e2[0].reshape(ER, 128), e2[1].reshape(ER, 128)
    sall = jnp.concatenate([s0, s1, s2], axis=0)
    dall = jnp.concatenate([d0, d1, d2], axis=0)
    Ws = jnp.stack([W0, W1, W2], axis=0)

    degf = _deg_call()(s0, d0, s1, d1, s2, d2)
    deg3 = jnp.reshape(degf, (NC * 6, NPD, DH))[:, :, 0]
    degT = jnp.transpose(deg3, (1, 0))
    y = _mm_call(degT[:N], x, Ws)
    acc = _scatter_call()(y, sall, dall)
    out = _comb_call(acc, degT[:N])
    return out


# speedup vs baseline: 1.0036x; 1.0036x over previous
import functools

import jax
import jax.numpy as jnp
from jax import lax
from jax.experimental import pallas as pl
from jax.experimental.pallas import tpu as pltpu, tpu_sc as plsc

N = 50000
D = 128
E = 200000
NREL = 3

NC = 2
NS = 16
L = 16

WSZ = 1024
NWIN = 196
E_PAD = NWIN * WSZ
PADV = N

NPD = 50048
SD = NPD // NS

CS = 4736
NCHUNK = 12
NPA = CS * NCHUNK
SA = CS // NS
NCC = NCHUNK // NC
ER = E_PAD // 128

@functools.cache
def _get_mesh():
    return plsc.VectorSubcoreMesh(core_axis_name="c", subcore_axis_name="s",
                                  num_cores=NC, num_subcores=NS)


WROWS = WSZ // 128


DH = 8
ZB = 4096


def _deg_body(s0, d0, s1, d1, s2, d2, deg_out, win, win8, ones_b, zeros_b,
              colbuf, table):
    c = lax.axis_index("c")
    s = lax.axis_index("s")
    w = s * NC + c

    one16 = jnp.full((L,), 1.0, jnp.float32)
    zero16 = jnp.zeros((L,), jnp.float32)

    def init_ones(i, _):
        ones_b[pl.ds(i * L, L)] = one16
        return _
    lax.fori_loop(0, 128 // L, init_ones, None)

    def init_zeros(i, _):
        zeros_b[pl.ds(i * L, L)] = zero16
        return _
    lax.fori_loop(0, ZB // L, init_zeros, None)

    base = s * SD * DH
    swords = SD * DH

    def zero_stripe():
        for t in range(swords // ZB):
            pltpu.sync_copy(zeros_b, table.at[pl.ds(base + t * ZB, ZB)])
        rem = swords - (swords // ZB) * ZB
        pltpu.sync_copy(zeros_b.at[pl.ds(0, rem)],
                        table.at[pl.ds(base + (swords // ZB) * ZB, rem)])

    zero_stripe()
    plsc.subcore_barrier()

    for h in range(6):
        e = (s0, d0, s1, d1, s2, d2)[h]

        def wbody(i, _):
            wi = w + NC * NS * i

            @pl.when(wi < NWIN)
            def _():
                pltpu.sync_copy(e.at[pl.ds(wi * WROWS, WROWS)], win)
                for g in range(WROWS):
                    def jbody(jj, _):
                        col = jj * L
                        win8[g, pl.ds(col, L)] = win[g, pl.ds(col, L)] * DH
                        return _
                    lax.fori_loop(0, 128 // L, jbody, None)
                    pltpu.sync_copy(ones_b, table.at[win8.at[g]], add=True)
            return _

        lax.fori_loop(0, (NWIN + NC * NS - 1) // (NC * NS), wbody, None)
        plsc.subcore_barrier()

        pltpu.sync_copy(table.at[pl.ds(base, swords)], colbuf)
        pltpu.sync_copy(
            colbuf, deg_out.at[pl.ds((c * 6 + h) * NPD * DH + base, swords)])

        if h < 5:
            zero_stripe()
        plsc.subcore_barrier()


@functools.cache
def _deg_call():
    return pl.kernel(
        _deg_body,
        out_type=jax.ShapeDtypeStruct((NC * 6 * NPD * DH,), jnp.float32),
        mesh=_get_mesh(),
        scratch_types=[
            pltpu.VMEM((WROWS, 128), jnp.int32),
            pltpu.VMEM((WROWS, 128), jnp.int32),
            pltpu.VMEM((128,), jnp.float32),
            pltpu.VMEM((ZB,), jnp.float32),
            pltpu.VMEM((SD * DH,), jnp.float32),
            pltpu.VMEM_SHARED((NPD * DH,), jnp.float32),
        ],
    )


def _scatter_body(y, esall, edall, acc, winS, winD,
                  gidx3, lidx3, rows, rows2, zrows, wb, sem, sem2, table):
    c = lax.axis_index("c")
    s = lax.axis_index("s")

    zero16 = jnp.zeros((L,), jnp.float32)

    def zinit(i, _):
        r0 = i // (D // L)
        col = (i % (D // L)) * L
        zrows[r0, pl.ds(col, L)] = zero16
        return _
    lax.fori_loop(0, 128 * (D // L), zinit, None)

    iota16 = lax.iota(jnp.int32, L)
    base = s * SA

    def pbody(p, _):
        r = p // NCC
        cc = p % NCC
        lo = (c + NC * cc) * CS
        erow = r * ER

        for t in range(SA // 128):
            pltpu.sync_copy(zrows, table.at[pl.ds(base + t * 128, 128)])
        pltpu.sync_copy(zrows.at[pl.ds(0, SA - (SA // 128) * 128)],
                        table.at[pl.ds(base + (SA // 128) * 128,
                                       SA - (SA // 128) * 128)])
        plsc.subcore_barrier()

        dump16 = CS + iota16
        roff = r * N

        def wbody(i, _):
            wi = s + NS * i

            @pl.when(wi < NWIN)
            def _():
                pltpu.sync_copy(
                    esall.at[pl.ds(erow + wi * WROWS, WROWS)], winS)
                pltpu.sync_copy(
                    edall.at[pl.ds(erow + wi * WROWS, WROWS)], winD)

                for g in range(WROWS):
                    def jbody(jj, _):
                        col = jj * L
                        src16 = winS[g, pl.ds(col, L)]
                        dst16 = winD[g, pl.ds(col, L)]
                        m = (dst16 >= lo) & (dst16 < lo + CS)
                        lidx3[g, pl.ds(col, L)] = jnp.where(
                            m, dst16 - lo, dump16)
                        gidx3[g, pl.ds(col, L)] = (
                            jnp.minimum(src16, N - 1) + roff)
                        return _
                    lax.fori_loop(0, 128 // L, jbody, None)

                bufs = (rows, rows2)
                sems = (sem, sem2)
                dprev = pltpu.async_copy(y.at[gidx3.at[0]], rows, sem)
                for g in range(WROWS):
                    dprev.wait()
                    if g + 1 < WROWS:
                        dprev = pltpu.async_copy(
                            y.at[gidx3.at[g + 1]], bufs[(g + 1) % 2],
                            sems[(g + 1) % 2])
                    pltpu.sync_copy(bufs[g % 2], table.at[lidx3.at[g]],
                                    add=True)
            return _

        lax.fori_loop(0, (NWIN + NS - 1) // NS, wbody, None)
        plsc.subcore_barrier()

        pltpu.sync_copy(table.at[pl.ds(base, SA)], wb)
        pltpu.sync_copy(wb, acc.at[r, pl.ds(lo + base, SA)])
        plsc.subcore_barrier()
        return _

    lax.fori_loop(0, NREL * NCC, pbody, None)


@functools.cache
def _scatter_call():
    return pl.kernel(
        _scatter_body,
        out_type=jax.ShapeDtypeStruct((NREL, NPA, D), jnp.float32),
        mesh=_get_mesh(),
        scratch_types=[
            pltpu.VMEM((WROWS, 128), jnp.int32),
            pltpu.VMEM((WROWS, 128), jnp.int32),
            pltpu.VMEM((WROWS, 128), jnp.int32),
            pltpu.VMEM((WROWS, 128), jnp.int32),
            pltpu.VMEM((128, D), jnp.float32),
            pltpu.VMEM((128, D), jnp.float32),
            pltpu.VMEM((128, D), jnp.float32),
            pltpu.VMEM((SA, D), jnp.float32),
            pltpu.SemaphoreType.DMA,
            pltpu.SemaphoreType.DMA,
            pltpu.VMEM_SHARED((CS + L, D), jnp.float32),
        ],
    )

MMBLK = 1000


def _make_mm():
    nb = N // MMBLK

    def gen_body(degT_ref, x_ref, w_ref, y_ref):
        r = pl.program_id(0)
        dp = degT_ref[...]
        c0 = dp[:, 0:1] + dp[:, 6:7]
        c1 = dp[:, 2:3] + dp[:, 8:9]
        c2 = dp[:, 4:5] + dp[:, 10:11]
        dsum = jnp.where(r == 0, c0, jnp.where(r == 1, c1, c2))
        norm = jnp.where(dsum > 0, lax.rsqrt(jnp.where(dsum > 0, dsum, 1.0)),
                         0.0)
        y_ref[...] = jnp.dot(x_ref[...] * norm, w_ref[0],
                             preferred_element_type=jnp.float32)

    return pl.pallas_call(
        gen_body,
        grid=(NREL, nb),
        in_specs=[
            pl.BlockSpec((MMBLK, 12), lambda r, b: (b, 0)),
            pl.BlockSpec((MMBLK, D), lambda r, b: (b, 0)),
            pl.BlockSpec((1, D, D), lambda r, b: (r, 0, 0)),
        ],
        out_specs=pl.BlockSpec((MMBLK, D), lambda r, b: (r * nb + b, 0)),
        out_shape=jax.ShapeDtypeStruct((NREL * N, D), jnp.float32),
    )


_mm_call = _make_mm()


def _comb_body(acc_ref, degT_ref, out_ref):
    dp = degT_ref[...]
    out = jnp.zeros((MMBLK, D), jnp.float32)
    for r in range(NREL):
        dsum = dp[:, 2 * r + 1:2 * r + 2] + dp[:, 6 + 2 * r + 1:6 + 2 * r + 2]
        norm = jnp.where(dsum > 0, lax.rsqrt(jnp.where(dsum > 0, dsum, 1.0)),
                         0.0)
        out = out + acc_ref[r] * norm
    out_ref[...] = out


_comb_call = pl.pallas_call(
    _comb_body,
    grid=(N // MMBLK,),
    in_specs=[
        pl.BlockSpec((NREL, MMBLK, D), lambda b: (0, b, 0)),
        pl.BlockSpec((MMBLK, 12), lambda b: (b, 0)),
    ],
    out_specs=pl.BlockSpec((MMBLK, D), lambda b: (b, 0)),
    out_shape=jax.ShapeDtypeStruct((N, D), jnp.float32),
)


def kernel(x, W0, W1, W2, edge_index_0, edge_index_1, edge_index_2):
    pad = ((0, 0), (0, E_PAD - E))
    e0 = jnp.pad(edge_index_0, pad, constant_values=PADV)
    e1 = jnp.pad(edge_index_1, pad, constant_values=PADV)
    e2 = jnp.pad(edge_index_2, pad, constant_values=PADV)
    s0, d0 = e0[0].reshape(ER, 128), e0[1].reshape(ER, 128)
    s1, d1 = e1[0].reshape(ER, 128), e1[1].reshape(ER, 128)
    s2, d2 = e2[0].reshape(ER, 128), e2[1].reshape(ER, 128)
    sall = jnp.concatenate([s0, s1, s2], axis=0)
    dall = jnp.concatenate([d0, d1, d2], axis=0)
    Ws = jnp.stack([W0, W1, W2], axis=0)

    degf = _deg_call()(s0, d0, s1, d1, s2, d2)
    deg3 = jnp.reshape(degf, (NC * 6, NPD, DH))[:, :, 0]
    degT = jnp.transpose(deg3, (1, 0))
    y = _mm_call(degT[:N], x, Ws)
    acc = _scatter_call()(y, sall, dall)
    out = _comb_call(acc, degT[:N])
    return out
